# baseline (speedup 1.0000x reference)
import functools

import jax
import jax.numpy as jnp
from jax import lax
from jax.experimental import pallas as pl
from jax.experimental.pallas import tpu as pltpu
from jax.experimental.pallas import tpu_sc as plsc

VOCAB = 1000
EMB = 60
STRIDE = 61
EMBP = 64
B = 16384
L = 20
LANES = 16

_INFO = plsc.get_sparse_core_info()
NC = _INFO.num_cores
NS = _INFO.num_subcores
NW = NC * NS
NBT = B // 128
UNITS = L * NBT
UPW = UNITS // NW
NGRP = 128 // LANES
NSUPER = UPW // 8


def _matmul_body(a_ref, x_ref, o_ref):
    o_ref[...] = jnp.dot(a_ref[...], x_ref[...],
                         preferred_element_type=jnp.float32)


def _propagate(A, x):
    return pl.pallas_call(
        _matmul_body,
        out_shape=jax.ShapeDtypeStruct((VOCAB, EMB), jnp.float32),
    )(A, x)


@functools.partial(
    pl.kernel,
    mesh=plsc.VectorSubcoreMesh(core_axis_name="c", subcore_axis_name="s"),
    out_type=jax.ShapeDtypeStruct((L, EMBP, B), jnp.float32),
    scratch_types=[
        pltpu.VMEM((VOCAB * STRIDE,), jnp.float32),
        pltpu.VMEM((2, 8, 128), jnp.int32),
        pltpu.VMEM((4, EMBP, 128), jnp.float32),
        [pltpu.SemaphoreType.DMA] * 4,
        [pltpu.SemaphoreType.DMA] * 2,
    ],
    compiler_params=pltpu.CompilerParams(needs_layout_passes=False),
)
def _gather(xg_hbm, ids_hbm, out_hbm, table_v, idx_v, stage_v, sem_w, sem_i):
    wid = lax.axis_index("s") * NC + lax.axis_index("c")
    u0 = wid * UPW

    def fire_idx(s, slot):
        n0 = u0 + 8 * s
        l = lax.div(n0, NBT)
        row0 = pl.multiple_of(lax.rem(n0, NBT), 8)
        pltpu.async_copy(
            ids_hbm.at[l, pl.ds(row0, 8)], idx_v.at[slot], sem_i[slot])

    def wait_idx(slot):
        pltpu.make_async_copy(
            ids_hbm.at[0, pl.ds(0, 8)], idx_v.at[slot], sem_i[slot]).wait()

    def drain_write(p):
        pltpu.make_async_copy(
            stage_v.at[p],
            out_hbm.at[0, pl.ds(0, EMBP), pl.ds(0, 128)],
            sem_w[p],
        ).wait()

    fire_idx(0, 0)
    pltpu.sync_copy(xg_hbm, table_v)

    def do_unit(l, bt, slot, j, p):
        def one_grp(g2, half):
            goff = pl.multiple_of((2 * g2 + half) * LANES, LANES)
            ptr0 = idx_v[slot, j, pl.ds(goff, LANES)]
            for e0 in range(0, EMB, 15):
                vals = [plsc.load_gather(table_v, [ptr0 + (e0 + t)])
                        for t in range(15)]
                for t in range(15):
                    stage_v[p, e0 + t, pl.ds(goff, LANES)] = vals[t]

        def grp(g2, carry):
            one_grp(g2, 0)
            one_grp(g2, 1)
            return carry

        lax.fori_loop(0, NGRP // 2, grp, 0)
        b_off = pl.multiple_of((bt + j) * 128, 128)
        pltpu.async_copy(
            stage_v.at[p],
            out_hbm.at[l, pl.ds(0, EMBP), pl.ds(b_off, 128)],
            sem_w[p],
        )

    def super_body(s, carry):
        q = lax.rem(s, 2)
        n0 = u0 + 8 * s
        l = lax.div(n0, NBT)
        bt0 = lax.rem(n0, NBT)

        @pl.when((s < NSUPER - 1) & (q == 0))
        def _():
            fire_idx(s + 1, 1)

        @pl.when((s < NSUPER - 1) & (q == 1))
        def _():
            fire_idx(s + 1, 0)

        @pl.when(q == 0)
        def _():
            wait_idx(0)

        @pl.when(q == 1)
        def _():
            wait_idx(1)

        for j in range(8):
            p = j % 4
            if j < 4:
                @pl.when(s > 0)
                def _():
                    drain_write(p)
            else:
                drain_write(p)
            do_unit(l, bt0, q, j, p)
        return carry

    lax.fori_loop(0, NSUPER, super_body, 0)
    for p in range(4):
        drain_write(p)


def kernel(A, x, movie_ids):
    xg = _propagate(A, x)
    xg_flat = jnp.pad(xg, ((0, 0), (0, STRIDE - EMB))).reshape(VOCAB * STRIDE)
    ids_sc = (movie_ids.astype(jnp.int32) * STRIDE).T
    p = _gather(xg_flat, ids_sc.reshape(L, NBT, 128))
    return p.transpose(2, 0, 1)[:, :, :EMB]
